# baseline (speedup 1.0000x reference)
import jax
import jax.numpy as jnp
from jax import lax
from jax.experimental import pallas as pl
from jax.experimental.pallas import tpu as pltpu
from jax.experimental.pallas import tpu_sc as plsc

B, N, H = 8, 128, 128
L = 16
QJ = N // 2
NCH = QJ // L


def _sc_er_agg_body(er_hbm, adj_hbm, red_hbm, er_vm, adj_vm, out_vm):
    wid = lax.axis_index("s")
    b = wid // 2
    q = wid % 2
    pltpu.sync_copy(adj_hbm.at[pl.ds(q * QJ, QJ), :], adj_vm)
    pltpu.sync_copy(er_hbm.at[b, :, pl.ds(q * QJ, QJ), :], er_vm)

    jl = [ch * L + lax.iota(jnp.int32, L) for ch in range(NCH)]
    cv = [jnp.full((L,), c, jnp.int32) for c in range(3)]
    zero = jnp.zeros((L,), jnp.float32)
    zf = jnp.zeros((L,), jnp.float32)
    iv0 = jnp.zeros((L,), jnp.int32)

    def body(_, st):
        accs, iv = st
        new = []
        for ch in range(NCH):
            m = plsc.load_gather(adj_vm, [jl[ch], iv])
            sel = m > 0
            row = []
            for c in range(3):
                e = plsc.load_gather(er_vm, [iv, jl[ch], cv[c]])
                row.append(accs[ch][c] + jnp.where(sel, e, zf))
            new.append(tuple(row))
        return (tuple(new), iv + 1)

    init = (tuple((zero, zero, zero) for _ in range(NCH)), iv0)
    accs, _ = lax.fori_loop(0, N, body, init, unroll=8)
    for ch in range(NCH):
        for c in range(3):
            out_vm[c, pl.ds(ch * L, L)] = accs[ch][c]

    for c in range(3):
        pltpu.sync_copy(out_vm.at[c], red_hbm.at[b, c, pl.ds(q * QJ, QJ)])


def _sc_er_agg(er, adj):
    mesh = plsc.VectorSubcoreMesh(core_axis_name="c", subcore_axis_name="s",
                                  num_cores=1)
    return pl.kernel(
        _sc_er_agg_body,
        mesh=mesh,
        compiler_params=pltpu.CompilerParams(use_tc_tiling_on_sc=False,
                                             needs_layout_passes=False),
        out_type=jax.ShapeDtypeStruct((B, 3, N), jnp.float32),
        scratch_types=[
            pltpu.VMEM((N, QJ, 3), jnp.float32),
            pltpu.VMEM((QJ, N), jnp.int32),
            pltpu.VMEM((3, QJ), jnp.float32),
        ],
    )(er, adj)


def _fused_kernel(adj_ref, ne_ref, red_ref, wmsg_ref, bmsg_ref,
                  wup_ref, bup_ref, gamma_ref, beta_ref, out_ref):
    f32 = jnp.float32
    dn = (((1,), (1,)), ((), ()))
    mask = (adj_ref[:] > 0).astype(f32)
    deg = jnp.sum(mask, axis=1, keepdims=True)
    bias_jh = deg * bmsg_ref[:]
    wn = wmsg_ref[:, :H]
    we = wmsg_ref[:, H:]
    wu1 = wup_ref[:, :H]
    wu2 = wup_ref[:, H:]
    bup = bup_ref[:]

    s = jnp.zeros((1, H), f32)
    s2 = jnp.zeros((1, H), f32)
    for b in range(B):
        ne_b = ne_ref[b]
        red_b = red_ref[b]
        term_b = lax.dot_general(red_b, we, (((0,), (1,)), ((), ())),
                                 preferred_element_type=f32)
        proj_b = lax.dot_general(ne_b, wn, dn,
                                 preferred_element_type=f32)
        msg_b = jnp.dot(mask, proj_b, preferred_element_type=f32) \
            + term_b + bias_jh
        up_b = lax.dot_general(ne_b, wu1, dn, preferred_element_type=f32) \
            + lax.dot_general(msg_b, wu2, dn, preferred_element_type=f32) \
            + bup
        up_b = jnp.maximum(up_b, 0.0)
        out_ref[b] = up_b
        s = s + jnp.sum(up_b, axis=0, keepdims=True)
        s2 = s2 + jnp.sum(up_b * up_b, axis=0, keepdims=True)

    inv_n = 1.0 / (B * N)
    mean = s * inv_n
    var = s2 * inv_n - mean * mean
    scale = lax.rsqrt(var + 1e-5) * gamma_ref[:]
    shift = beta_ref[:] - mean * scale
    for b in range(B):
        out_ref[b] = out_ref[b] * scale + shift + ne_ref[b]


def kernel(node_embeddings, edge_relations, adjacency, W_msg, b_msg,
           W_up, b_up, bn_gamma, bn_beta):
    adj = adjacency.astype(jnp.int32)
    red = _sc_er_agg(edge_relations, adj)
    bmsg = b_msg.reshape(1, H)
    bup = b_up.reshape(1, H)
    gamma = bn_gamma.reshape(1, H)
    beta = bn_beta.reshape(1, H)
    return pl.pallas_call(
        _fused_kernel,
        out_shape=jax.ShapeDtypeStruct((B, N, H), jnp.float32),
    )(adj, node_embeddings, red, W_msg, bmsg, W_up, bup, gamma, beta)


# speedup vs baseline: 4.1452x; 4.1452x over previous
import jax
import jax.numpy as jnp
from jax import lax
from jax.experimental import pallas as pl
from jax.experimental.pallas import tpu as pltpu
from jax.experimental.pallas import tpu_sc as plsc

B, N, H = 8, 128, 128
L = 16
QJ = N // 2
NCH = QJ // L


def _sc_er_agg_body(er_hbm, adj_hbm, red_hbm, er_vm, adj_vm, out_vm):
    wid = lax.axis_index("s")
    b = wid // 2
    q = wid % 2
    pltpu.sync_copy(adj_hbm.at[pl.ds(q * QJ, QJ), :], adj_vm)
    pltpu.sync_copy(er_hbm.at[b, :, pl.ds(q * QJ * 3, QJ * 3)], er_vm)

    jl = [ch * L + lax.iota(jnp.int32, L) for ch in range(NCH)]
    jc = [[j3 * 3 + c for c in range(3)] for j3 in jl]
    zero = jnp.zeros((L,), jnp.float32)
    zf = jnp.zeros((L,), jnp.float32)
    iv0 = jnp.zeros((L,), jnp.int32)

    def body(_, st):
        accs, iv = st
        new = []
        for ch in range(NCH):
            m = plsc.load_gather(adj_vm, [jl[ch], iv])
            sel = m > 0
            row = []
            for c in range(3):
                e = plsc.load_gather(er_vm, [iv, jc[ch][c]])
                row.append(accs[ch][c] + jnp.where(sel, e, zf))
            new.append(tuple(row))
        return (tuple(new), iv + 1)

    init = (tuple((zero, zero, zero) for _ in range(NCH)), iv0)
    accs, _ = lax.fori_loop(0, N, body, init, unroll=8)
    for ch in range(NCH):
        for c in range(3):
            out_vm[c, pl.ds(ch * L, L)] = accs[ch][c]

    for c in range(3):
        pltpu.sync_copy(out_vm.at[c], red_hbm.at[b, c, pl.ds(q * QJ, QJ)])


def _sc_er_agg(er, adj):
    mesh = plsc.VectorSubcoreMesh(core_axis_name="c", subcore_axis_name="s",
                                  num_cores=1)
    return pl.kernel(
        _sc_er_agg_body,
        mesh=mesh,
        compiler_params=pltpu.CompilerParams(use_tc_tiling_on_sc=False,
                                             needs_layout_passes=False),
        out_type=jax.ShapeDtypeStruct((B, 3, N), jnp.float32),
        scratch_types=[
            pltpu.VMEM((N, QJ * 3), jnp.float32),
            pltpu.VMEM((QJ, N), jnp.int32),
            pltpu.VMEM((3, QJ), jnp.float32),
        ],
    )(er, adj)


def _fused_kernel(adj_ref, ne_ref, red_ref, wmsg_ref, bmsg_ref,
                  wup_ref, bup_ref, gamma_ref, beta_ref, out_ref):
    f32 = jnp.float32
    dn = (((1,), (1,)), ((), ()))
    mask = (adj_ref[:] > 0).astype(f32)
    deg = jnp.sum(mask, axis=1, keepdims=True)
    bias_jh = deg * bmsg_ref[:]
    wn = wmsg_ref[:, :H]
    we = wmsg_ref[:, H:]
    wu1 = wup_ref[:, :H]
    wu2 = wup_ref[:, H:]
    bup = bup_ref[:]

    s = jnp.zeros((1, H), f32)
    s2 = jnp.zeros((1, H), f32)
    for b in range(B):
        ne_b = ne_ref[b]
        red_b = red_ref[b]
        term_b = lax.dot_general(red_b, we, (((0,), (1,)), ((), ())),
                                 preferred_element_type=f32)
        proj_b = lax.dot_general(ne_b, wn, dn,
                                 preferred_element_type=f32)
        msg_b = jnp.dot(mask, proj_b, preferred_element_type=f32) \
            + term_b + bias_jh
        up_b = lax.dot_general(ne_b, wu1, dn, preferred_element_type=f32) \
            + lax.dot_general(msg_b, wu2, dn, preferred_element_type=f32) \
            + bup
        up_b = jnp.maximum(up_b, 0.0)
        out_ref[b] = up_b
        s = s + jnp.sum(up_b, axis=0, keepdims=True)
        s2 = s2 + jnp.sum(up_b * up_b, axis=0, keepdims=True)

    inv_n = 1.0 / (B * N)
    mean = s * inv_n
    var = s2 * inv_n - mean * mean
    scale = lax.rsqrt(var + 1e-5) * gamma_ref[:]
    shift = beta_ref[:] - mean * scale
    for b in range(B):
        out_ref[b] = out_ref[b] * scale + shift + ne_ref[b]


def kernel(node_embeddings, edge_relations, adjacency, W_msg, b_msg,
           W_up, b_up, bn_gamma, bn_beta):
    adj = adjacency.astype(jnp.int32)
    er3 = edge_relations.reshape(B, N, N * 3)
    red = _sc_er_agg(er3, adj)
    bmsg = b_msg.reshape(1, H)
    bup = b_up.reshape(1, H)
    gamma = bn_gamma.reshape(1, H)
    beta = bn_beta.reshape(1, H)
    return pl.pallas_call(
        _fused_kernel,
        out_shape=jax.ShapeDtypeStruct((B, N, H), jnp.float32),
    )(adj, node_embeddings, red, W_msg, bmsg, W_up, bup, gamma, beta)


# speedup vs baseline: 5.1531x; 1.2431x over previous
import jax
import jax.numpy as jnp
from jax import lax
from jax.experimental import pallas as pl
from jax.experimental.pallas import tpu as pltpu
from jax.experimental.pallas import tpu_sc as plsc

B, N, H = 8, 128, 128
L = 16
QJ = N // 4
NCH = QJ // L


def _sc_er_agg_body(er_hbm, adj_hbm, red_hbm, er_vm, adj_vm, out_vm):
    wid = lax.axis_index("s") * 2 + lax.axis_index("c")
    b = wid // 4
    q = wid % 4
    pltpu.sync_copy(adj_hbm.at[pl.ds(q * QJ, QJ), :], adj_vm)
    pltpu.sync_copy(er_hbm.at[b, :, pl.ds(q * QJ * 3, QJ * 3)], er_vm)

    jl = [ch * L + lax.iota(jnp.int32, L) for ch in range(NCH)]
    jc = [[j3 * 3 + c for c in range(3)] for j3 in jl]
    zero = jnp.zeros((L,), jnp.float32)
    zf = jnp.zeros((L,), jnp.float32)
    iv0 = jnp.zeros((L,), jnp.int32)

    def body(_, st):
        accs, iv = st
        new = []
        for ch in range(NCH):
            m = plsc.load_gather(adj_vm, [jl[ch], iv])
            sel = m > 0
            row = []
            for c in range(3):
                e = plsc.load_gather(er_vm, [iv, jc[ch][c]])
                row.append(accs[ch][c] + jnp.where(sel, e, zf))
            new.append(tuple(row))
        return (tuple(new), iv + 1)

    init = (tuple((zero, zero, zero) for _ in range(NCH)), iv0)
    accs, _ = lax.fori_loop(0, N, body, init, unroll=4)
    for ch in range(NCH):
        for c in range(3):
            out_vm[c, pl.ds(ch * L, L)] = accs[ch][c]

    for c in range(3):
        pltpu.sync_copy(out_vm.at[c], red_hbm.at[b, c, pl.ds(q * QJ, QJ)])


def _sc_er_agg(er, adj):
    mesh = plsc.VectorSubcoreMesh(core_axis_name="c", subcore_axis_name="s")
    return pl.kernel(
        _sc_er_agg_body,
        mesh=mesh,
        compiler_params=pltpu.CompilerParams(use_tc_tiling_on_sc=False,
                                             needs_layout_passes=False),
        out_type=jax.ShapeDtypeStruct((B, 3, N), jnp.float32),
        scratch_types=[
            pltpu.VMEM((N, QJ * 3), jnp.float32),
            pltpu.VMEM((QJ, N), jnp.int32),
            pltpu.VMEM((3, QJ), jnp.float32),
        ],
    )(er, adj)


def _fused_kernel(adj_ref, ne_ref, red_ref, wmsg_ref, bmsg_ref,
                  wup_ref, bup_ref, gamma_ref, beta_ref, out_ref):
    f32 = jnp.float32
    dn = (((1,), (1,)), ((), ()))
    mask = (adj_ref[:] > 0).astype(f32)
    deg = jnp.sum(mask, axis=1, keepdims=True)
    bias_jh = deg * bmsg_ref[:]
    wn = wmsg_ref[:, :H]
    we = wmsg_ref[:, H:]
    wu1 = wup_ref[:, :H]
    wu2 = wup_ref[:, H:]
    bup = bup_ref[:]

    s = jnp.zeros((1, H), f32)
    s2 = jnp.zeros((1, H), f32)
    for b in range(B):
        ne_b = ne_ref[b]
        red_b = red_ref[b]
        term_b = lax.dot_general(red_b, we, (((0,), (1,)), ((), ())),
                                 preferred_element_type=f32)
        proj_b = lax.dot_general(ne_b, wn, dn,
                                 preferred_element_type=f32)
        msg_b = jnp.dot(mask, proj_b, preferred_element_type=f32) \
            + term_b + bias_jh
        up_b = lax.dot_general(ne_b, wu1, dn, preferred_element_type=f32) \
            + lax.dot_general(msg_b, wu2, dn, preferred_element_type=f32) \
            + bup
        up_b = jnp.maximum(up_b, 0.0)
        out_ref[b] = up_b
        s = s + jnp.sum(up_b, axis=0, keepdims=True)
        s2 = s2 + jnp.sum(up_b * up_b, axis=0, keepdims=True)

    inv_n = 1.0 / (B * N)
    mean = s * inv_n
    var = s2 * inv_n - mean * mean
    scale = lax.rsqrt(var + 1e-5) * gamma_ref[:]
    shift = beta_ref[:] - mean * scale
    for b in range(B):
        out_ref[b] = out_ref[b] * scale + shift + ne_ref[b]


def kernel(node_embeddings, edge_relations, adjacency, W_msg, b_msg,
           W_up, b_up, bn_gamma, bn_beta):
    adj = adjacency.astype(jnp.int32)
    er3 = edge_relations.reshape(B, N, N * 3)
    red = _sc_er_agg(er3, adj)
    bmsg = b_msg.reshape(1, H)
    bup = b_up.reshape(1, H)
    gamma = bn_gamma.reshape(1, H)
    beta = bn_beta.reshape(1, H)
    return pl.pallas_call(
        _fused_kernel,
        out_shape=jax.ShapeDtypeStruct((B, N, H), jnp.float32),
    )(adj, node_embeddings, red, W_msg, bmsg, W_up, bup, gamma, beta)
